# initial kernel scaffold (unmeasured)
import jax
import jax.numpy as jnp
from jax import lax
from jax.experimental import pallas as pl
from jax.experimental.pallas import tpu as pltpu

N_DEV = 8


def kernel(x, w_mat):
    m, k_per = x.shape
    k_full, n = w_mat.shape
    m_per = m // N_DEV
    assert m_per * N_DEV == m and k_per * N_DEV == k_full

    def body(x_ref, w_ref, out_ref, xg_ref, send_sems, recv_sems):
        kstep = pl.program_id(0)
        my = lax.axis_index("i")

        def make_send(t):
            return pltpu.make_async_remote_copy(
                src_ref=x_ref.at[pl.ds(t * m_per, m_per), :],
                dst_ref=xg_ref.at[my],
                send_sem=send_sems.at[t],
                recv_sem=recv_sems.at[my],
                device_id=(t,),
                device_id_type=pl.DeviceIdType.MESH,
            )

        @pl.when(kstep == 0)
        def _():
            barrier = pltpu.get_barrier_semaphore()
            for t in range(N_DEV):
                @pl.when(my != t)
                def _(t=t):
                    pl.semaphore_signal(
                        barrier, inc=1,
                        device_id=(t,), device_id_type=pl.DeviceIdType.MESH,
                    )
            pl.semaphore_wait(barrier, N_DEV - 1)

            xg_ref[my] = x_ref[pl.ds(my * m_per, m_per), :]
            for t in range(N_DEV):
                @pl.when(my != t)
                def _(t=t):
                    make_send(t).start()

        @pl.when(kstep != my)
        def _():
            recv = pltpu.make_async_remote_copy(
                src_ref=x_ref.at[pl.ds(0, m_per), :],
                dst_ref=xg_ref.at[kstep],
                send_sem=send_sems.at[0],
                recv_sem=recv_sems.at[kstep],
                device_id=(0,),
                device_id_type=pl.DeviceIdType.MESH,
            )
            recv.wait_recv()

        partial = jnp.dot(
            xg_ref[kstep], w_ref[...], preferred_element_type=jnp.float32
        )

        @pl.when(kstep == 0)
        def _():
            out_ref[...] = partial

        @pl.when(jnp.logical_and(kstep > 0, kstep < N_DEV - 1))
        def _():
            out_ref[...] += partial

        @pl.when(kstep == N_DEV - 1)
        def _():
            y = out_ref[...] + partial
            out_ref[...] = y * jax.nn.sigmoid(y)
            for t in range(N_DEV):
                @pl.when(my != t)
                def _(t=t):
                    make_send(t).wait_send()

    return pl.pallas_call(
        body,
        grid=(N_DEV,),
        in_specs=[
            pl.BlockSpec((m, k_per), lambda k: (0, 0)),
            pl.BlockSpec((k_per, n), lambda k: (k, 0)),
        ],
        out_specs=pl.BlockSpec((m_per, n), lambda k: (0, 0)),
        out_shape=jax.ShapeDtypeStruct((m_per, n), jnp.float32),
        scratch_shapes=[
            pltpu.VMEM((N_DEV, m_per, k_per), jnp.float32),
            pltpu.SemaphoreType.DMA((N_DEV,)),
            pltpu.SemaphoreType.DMA((N_DEV,)),
        ],
        compiler_params=pltpu.CompilerParams(
            collective_id=0,
            dimension_semantics=("arbitrary",),
        ),
    )(x, w_mat)


# baseline (device time: 114242 ns/iter reference)
import jax
import jax.numpy as jnp
from jax import lax
from jax.experimental import pallas as pl
from jax.experimental.pallas import tpu as pltpu

N_DEV = 8
BN = 2048


def kernel(x, w_mat):
    m, k_per = x.shape
    k_full, n = w_mat.shape
    m_per = m // N_DEV
    nb = n // BN
    assert m_per * N_DEV == m and k_per * N_DEV == k_full and nb * BN == n

    def body(x_ref, w_ref, out_ref, xg_ref, acc_ref, send_sems, recv_sems):
        kstep = pl.program_id(0)
        j = pl.program_id(1)
        my = lax.axis_index("i")

        def make_send(t):
            return pltpu.make_async_remote_copy(
                src_ref=x_ref.at[pl.ds(t * m_per, m_per), :],
                dst_ref=xg_ref.at[my],
                send_sem=send_sems.at[t],
                recv_sem=recv_sems.at[my],
                device_id=(t,),
                device_id_type=pl.DeviceIdType.MESH,
            )

        @pl.when(jnp.logical_and(kstep == 0, j == 0))
        def _():
            barrier = pltpu.get_barrier_semaphore()
            for t in range(N_DEV):
                @pl.when(my != t)
                def _(t=t):
                    pl.semaphore_signal(
                        barrier, inc=1,
                        device_id=(t,), device_id_type=pl.DeviceIdType.MESH,
                    )
            pl.semaphore_wait(barrier, N_DEV - 1)

            xg_ref[my] = x_ref[pl.ds(my * m_per, m_per), :]
            for t in range(N_DEV):
                @pl.when(my != t)
                def _(t=t):
                    make_send(t).start()

        @pl.when(jnp.logical_and(j == 0, kstep != my))
        def _():
            recv = pltpu.make_async_remote_copy(
                src_ref=x_ref.at[pl.ds(0, m_per), :],
                dst_ref=xg_ref.at[kstep],
                send_sem=send_sems.at[0],
                recv_sem=recv_sems.at[kstep],
                device_id=(0,),
                device_id_type=pl.DeviceIdType.MESH,
            )
            recv.wait_recv()

        partial = jnp.dot(
            xg_ref[kstep], w_ref[...], preferred_element_type=jnp.float32
        )
        jtile = pl.ds(j * BN, BN)

        @pl.when(kstep == 0)
        def _():
            acc_ref[:, jtile] = partial

        @pl.when(jnp.logical_and(kstep > 0, kstep < N_DEV - 1))
        def _():
            acc_ref[:, jtile] += partial

        @pl.when(kstep == N_DEV - 1)
        def _():
            y = acc_ref[:, jtile] + partial
            out_ref[...] = y * jax.nn.sigmoid(y)

        @pl.when(jnp.logical_and(kstep == N_DEV - 1, j == nb - 1))
        def _():
            for t in range(N_DEV):
                @pl.when(my != t)
                def _(t=t):
                    make_send(t).wait_send()

    return pl.pallas_call(
        body,
        grid=(N_DEV, nb),
        in_specs=[
            pl.BlockSpec((m, k_per), lambda k, j: (0, 0)),
            pl.BlockSpec((k_per, BN), lambda k, j: (k, j)),
        ],
        out_specs=pl.BlockSpec(
            (m_per, BN),
            lambda k, j: (0, jnp.where(k == N_DEV - 1, j, 0)),
        ),
        out_shape=jax.ShapeDtypeStruct((m_per, n), jnp.float32),
        scratch_shapes=[
            pltpu.VMEM((N_DEV, m_per, k_per), jnp.float32),
            pltpu.VMEM((m_per, n), jnp.float32),
            pltpu.SemaphoreType.DMA((N_DEV,)),
            pltpu.SemaphoreType.DMA((N_DEV,)),
        ],
        compiler_params=pltpu.CompilerParams(
            collective_id=0,
            dimension_semantics=("arbitrary", "arbitrary"),
            vmem_limit_bytes=60 * 1024 * 1024,
        ),
    )(x, w_mat)


# device time: 91276 ns/iter; 1.2516x vs baseline; 1.2516x over previous
import numpy as np

import jax
import jax.numpy as jnp
from jax import lax
from jax.experimental import pallas as pl
from jax.experimental.pallas import tpu as pltpu

N_DEV = 8
N_PAIR = 4
BN = 2048

_COORDS = np.array([
    (0, 0, 0), (1, 0, 0), (1, 1, 0), (0, 1, 0),
    (0, 0, 1), (1, 0, 1), (1, 1, 1), (0, 1, 1),
])
_DIST = np.abs(_COORDS[:, None, :] - _COORDS[None, :, :]).sum(-1)

_SEND_ORDER = np.zeros((N_DEV, N_DEV - 1), dtype=np.int64)
for s in range(N_DEV):
    other_plane = (_COORDS[:, 2] != _COORDS[s, 2]).astype(np.float64)
    key = other_plane * 100.0 + _DIST[s] * 10.0 + np.arange(N_DEV) * 1e-3
    key[s ^ 1] = -1.0
    key[s] = 1e9
    _SEND_ORDER[s] = np.argsort(key)[:-1]

_PAIR_PERM = np.zeros((N_DEV, N_PAIR), dtype=np.int64)
for t in range(N_DEV):
    key = np.array([
        (_COORDS[2 * p, 2] != _COORDS[t, 2]) * 100.0
        + _DIST[t, 2 * p:2 * p + 2].max() * 10.0 + p * 1e-3
        for p in range(N_PAIR)
    ])
    key[t // 2] = -1.0
    _PAIR_PERM[t] = np.argsort(key)


def kernel(x, w_mat):
    m, k_per = x.shape
    k_full, n = w_mat.shape
    m_per = m // N_DEV
    nb = n // BN
    kp = 2 * k_per
    assert m_per * N_DEV == m and k_per * N_DEV == k_full and nb * BN == n

    def body(perm_ref, sends_ref, x_ref, w_ref, out_ref, xb_ref,
             xg_ref, acc_ref, send_sems, recv_sems):
        kstep = pl.program_id(0)
        j = pl.program_id(1)
        my = lax.axis_index("i")

        def send_rdma(idx):
            t = sends_ref[idx]
            return pltpu.make_async_remote_copy(
                src_ref=xb_ref.at[pl.ds(t * m_per, m_per), :],
                dst_ref=xg_ref.at[:, pl.ds(my * k_per, k_per)],
                send_sem=send_sems.at[idx],
                recv_sem=recv_sems.at[my],
                device_id=(t,),
                device_id_type=pl.DeviceIdType.MESH,
            )

        @pl.when(jnp.logical_and(kstep == 0, j == 0))
        def _():
            barrier = pltpu.get_barrier_semaphore()
            for t in range(N_DEV):
                @pl.when(my != t)
                def _(t=t):
                    pl.semaphore_signal(
                        barrier, inc=1,
                        device_id=(t,), device_id_type=pl.DeviceIdType.MESH,
                    )
            pl.semaphore_wait(barrier, N_DEV - 1)

            xb_ref[...] = x_ref[...].astype(jnp.bfloat16)
            xg_ref[:, pl.ds(my * k_per, k_per)] = xb_ref[
                pl.ds(my * m_per, m_per), :]
            for idx in range(N_DEV - 1):
                send_rdma(idx).start()

        pair = perm_ref[kstep]

        @pl.when(j == 0)
        def _():
            for half in range(2):
                src = 2 * pair + half

                @pl.when(src != my)
                def _(src=src):
                    recv = pltpu.make_async_remote_copy(
                        src_ref=xb_ref.at[pl.ds(0, m_per), :],
                        dst_ref=xg_ref.at[:, pl.ds(src * k_per, k_per)],
                        send_sem=send_sems.at[0],
                        recv_sem=recv_sems.at[src],
                        device_id=(0,),
                        device_id_type=pl.DeviceIdType.MESH,
                    )
                    recv.wait_recv()

        partial = jnp.dot(
            xg_ref[:, pl.ds(pair * kp, kp)].astype(jnp.float32),
            w_ref[...],
            preferred_element_type=jnp.float32,
        )
        jtile = pl.ds(j * BN, BN)

        @pl.when(kstep == 0)
        def _():
            acc_ref[:, jtile] = partial.astype(jnp.bfloat16)

        @pl.when(jnp.logical_and(kstep > 0, kstep < N_PAIR - 1))
        def _():
            acc_ref[:, jtile] = (
                acc_ref[:, jtile].astype(jnp.float32) + partial
            ).astype(jnp.bfloat16)

        @pl.when(kstep == N_PAIR - 1)
        def _():
            y = acc_ref[:, jtile].astype(jnp.float32) + partial
            out_ref[...] = y * jax.nn.sigmoid(y)

        @pl.when(jnp.logical_and(kstep == N_PAIR - 1, j == nb - 1))
        def _():
            for idx in range(N_DEV - 1):
                send_rdma(idx).wait_send()

    grid_spec = pltpu.PrefetchScalarGridSpec(
        num_scalar_prefetch=2,
        grid=(N_PAIR, nb),
        in_specs=[
            pl.BlockSpec((m, k_per), lambda k, j, perm, sends: (0, 0)),
            pl.BlockSpec((kp, BN), lambda k, j, perm, sends: (perm[k], j)),
        ],
        out_specs=pl.BlockSpec(
            (m_per, BN),
            lambda k, j, perm, sends: (0, jnp.where(k == N_PAIR - 1, j, 0)),
        ),
        scratch_shapes=[
            pltpu.VMEM((m, k_per), jnp.bfloat16),
            pltpu.VMEM((m_per, k_full), jnp.bfloat16),
            pltpu.VMEM((m_per, n), jnp.bfloat16),
            pltpu.SemaphoreType.DMA((N_DEV - 1,)),
            pltpu.SemaphoreType.DMA((N_DEV,)),
        ],
    )

    my = lax.axis_index("i")
    perm = jnp.asarray(_PAIR_PERM, dtype=jnp.int32)[my]
    sends = jnp.asarray(_SEND_ORDER, dtype=jnp.int32)[my]
    return pl.pallas_call(
        body,
        grid_spec=grid_spec,
        out_shape=jax.ShapeDtypeStruct((m_per, n), jnp.float32),
        compiler_params=pltpu.CompilerParams(
            collective_id=0,
            dimension_semantics=("arbitrary", "arbitrary"),
            vmem_limit_bytes=62 * 1024 * 1024,
        ),
    )(perm, sends, x, w_mat)
